# baseline (speedup 1.0000x reference)
import functools
import jax
import jax.numpy as jnp
from jax import lax
from jax.experimental import pallas as pl
from jax.experimental.pallas import tpu as pltpu
from jax.experimental.pallas import tpu_sc as plsc

N = 20000
MAX_DET = 300
CONF_THRES = 0.25
IOU_THRES = 0.5

L = 16
NSUB = 16
SLICE = 1264
CH = SLICE // L
NPAD = NSUB * SLICE
OUTW = 8 * 304
OROW = 304


def _sc_nms(x1h, y1h, x2h, y2h, obh, brh, outh,
            sx1, sy1, sx2, sy2, sob, sbr,
            bx1, by1, bx2, by2, bar, bsc, bidx,
            recv, allv, outv, stage):
    c = lax.axis_index("c")
    s = lax.axis_index("s")
    is_writer = jnp.logical_and(c == 0, s == 0)
    base = s * SLICE
    lane = lax.iota(jnp.int32, L)
    lanef = lane.astype(jnp.float32)
    zeros16 = jnp.zeros((L,), jnp.float32)
    neg_inf = jnp.float32(-jnp.inf)

    pltpu.sync_copy(x1h.at[pl.ds(base, SLICE)], sx1)
    pltpu.sync_copy(y1h.at[pl.ds(base, SLICE)], sy1)
    pltpu.sync_copy(x2h.at[pl.ds(base, SLICE)], sx2)
    pltpu.sync_copy(y2h.at[pl.ds(base, SLICE)], sy2)
    pltpu.sync_copy(obh.at[pl.ds(base, SLICE)], sob)
    pltpu.sync_copy(brh.at[pl.ds(base, SLICE)], sbr)

    def _pre(k, carry):
        bsc[pl.ds(k * L, L)] = jnp.full((L,), neg_inf)
        return carry

    lax.fori_loop(0, CH, _pre, 0)

    def _zo(k, carry):
        outv[pl.ds(k * L, L)] = zeros16
        return carry

    lax.fori_loop(0, OUTW // L, _zo, 0)

    basef = base.astype(jnp.float32)

    def _cmp(j, off):
        o = sob[pl.ds(j * L, L)]
        b = sbr[pl.ds(j * L, L)]
        sco = jnp.clip(o, 0.0, 1.0) * jnp.clip(b, 0.0, 1.0)
        alive = sco >= CONF_THRES
        x1v = sx1[pl.ds(j * L, L)]
        y1v = sy1[pl.ds(j * L, L)]
        x2v = sx2[pl.ds(j * L, L)]
        y2v = sy2[pl.ds(j * L, L)]
        ar = (x2v - x1v) * (y2v - y1v)
        idxf = basef + (j * L).astype(jnp.float32) + lanef
        cs = jnp.cumsum(alive.astype(jnp.int32))
        tgt = off + cs - 1
        plsc.store_scatter(bx1, [tgt], x1v, mask=alive)
        plsc.store_scatter(by1, [tgt], y1v, mask=alive)
        plsc.store_scatter(bx2, [tgt], x2v, mask=alive)
        plsc.store_scatter(by2, [tgt], y2v, mask=alive)
        plsc.store_scatter(bar, [tgt], ar, mask=alive)
        plsc.store_scatter(bsc, [tgt], sco, mask=alive)
        plsc.store_scatter(bidx, [tgt], idxf, mask=alive)
        return off + jnp.max(cs)

    na = lax.fori_loop(0, CH, _cmp, jnp.int32(0))
    nch = (na + (L - 1)) // L

    def _iter(i, carry):
        wx1, wy1, wx2, wy2 = carry
        wa = (wx2 - wx1) * (wy2 - wy1)

        init = (jnp.full((L,), neg_inf), zeros16)

        def _pass(j, st):
            bs, bp = st
            x1v = bx1[pl.ds(j * L, L)]
            y1v = by1[pl.ds(j * L, L)]
            x2v = bx2[pl.ds(j * L, L)]
            y2v = by2[pl.ds(j * L, L)]
            scv = bsc[pl.ds(j * L, L)]
            arv = (x2v - x1v) * (y2v - y1v)
            xx1 = jnp.maximum(wx1, x1v)
            yy1 = jnp.maximum(wy1, y1v)
            xx2 = jnp.minimum(wx2, x2v)
            yy2 = jnp.minimum(wy2, y2v)
            inter = jnp.maximum(xx2 - xx1, 0.0) * jnp.maximum(yy2 - yy1, 0.0)
            iou = inter / jnp.maximum(wa + arv - inter, 1e-9)
            sc2 = jnp.where(iou >= IOU_THRES, neg_inf, scv)
            bsc[pl.ds(j * L, L)] = sc2
            posv = (j * L).astype(jnp.float32) + lanef
            upd = sc2 > bs
            bs = jnp.where(upd, sc2, bs)
            bp = jnp.where(upd, posv, bp)
            return (bs, bp)

        bs, bp = plsc.parallel_loop(0, nch, unroll=4, carry=init)(_pass)
        m_loc = jnp.max(bs)
        pos_w = jnp.min(jnp.where(bs == m_loc, bp, jnp.float32(3e7)))
        rel = jnp.full((L,), 0, jnp.int32) + pos_w.astype(jnp.int32)
        idxf = plsc.load_gather(bidx, [rel])
        gx1 = plsc.load_gather(bx1, [rel])
        gy1 = plsc.load_gather(by1, [rel])
        gx2 = plsc.load_gather(bx2, [rel])
        gy2 = plsc.load_gather(by2, [rel])
        rec = jnp.where(lane == 0, m_loc,
              jnp.where(lane == 1, idxf,
              jnp.where(lane == 2, gx1,
              jnp.where(lane == 3, gy1,
              jnp.where(lane == 4, gx2,
              jnp.where(lane == 5, gy2, 0.0))))))
        recv[...] = rec
        ph = lax.rem(i, 2) + 6
        pltpu.sync_copy(recv, stage.at[ph, s])
        plsc.subcore_barrier()
        pltpu.sync_copy(stage.at[ph], allv)

        def g(f):
            return plsc.load_gather(allv, [lane, jnp.full((L,), f, jnp.int32)])

        sc16 = g(0)
        id16 = g(1)
        m = jnp.max(sc16)
        wi = jnp.min(jnp.where(sc16 == m, id16, jnp.float32(3e7)))
        sel2 = jnp.logical_and(sc16 == m, id16 == wi)

        def pick(v):
            return jnp.max(jnp.where(sel2, v, jnp.float32(-1e30)))

        nx1 = pick(g(2))
        ny1 = pick(g(3))
        nx2 = pick(g(4))
        ny2 = pick(g(5))
        ok = m > neg_inf
        nx1 = jnp.where(ok, nx1, jnp.float32(2e9))
        ny1 = jnp.where(ok, ny1, jnp.float32(2e9))
        nx2 = jnp.where(ok, nx2, jnp.float32(-2e9))
        ny2 = jnp.where(ok, ny2, jnp.float32(-2e9))
        wm = jnp.logical_and(jnp.logical_and(is_writer, ok), lane == 0)
        mval = jnp.where(ok, m, jnp.float32(0.0))
        for fld, v in enumerate((nx1, ny1, nx2, ny2, mval)):
            plsc.store_scatter(
                outv, [jnp.full((L,), fld * OROW, jnp.int32) + i],
                v + zeros16, mask=wm)
        return (nx1, ny1, nx2, ny2)

    carry0 = (jnp.float32(2e9), jnp.float32(2e9),
              jnp.float32(-2e9), jnp.float32(-2e9))
    lax.fori_loop(0, MAX_DET, _iter, carry0)

    @pl.when(is_writer)
    def _flush():
        pltpu.sync_copy(outv, outh)


@jax.jit
def _run(x1, y1, x2, y2, ob, br):
    mesh = plsc.VectorSubcoreMesh(core_axis_name="c", subcore_axis_name="s")
    f = pl.kernel(
        _sc_nms,
        mesh=mesh,
        out_type=jax.ShapeDtypeStruct((OUTW,), jnp.float32),
        compiler_params=pltpu.CompilerParams(needs_layout_passes=False),
        scratch_types=[
            pltpu.VMEM((SLICE,), jnp.float32),
            pltpu.VMEM((SLICE,), jnp.float32),
            pltpu.VMEM((SLICE,), jnp.float32),
            pltpu.VMEM((SLICE,), jnp.float32),
            pltpu.VMEM((SLICE,), jnp.float32),
            pltpu.VMEM((SLICE,), jnp.float32),
            pltpu.VMEM((SLICE,), jnp.float32),
            pltpu.VMEM((SLICE,), jnp.float32),
            pltpu.VMEM((SLICE,), jnp.float32),
            pltpu.VMEM((SLICE,), jnp.float32),
            pltpu.VMEM((SLICE,), jnp.float32),
            pltpu.VMEM((SLICE,), jnp.float32),
            pltpu.VMEM((SLICE,), jnp.float32),
            pltpu.VMEM((L,), jnp.float32),
            pltpu.VMEM((NSUB, L), jnp.float32),
            pltpu.VMEM((OUTW,), jnp.float32),
            pltpu.VMEM_SHARED((8, NSUB, L), jnp.float32),
        ],
    )
    return f(x1, y1, x2, y2, ob, br)


def kernel(boxes, objectness, breed_conf):
    pad = NPAD - N
    x1 = jnp.pad(boxes[:, 0], (0, pad))
    y1 = jnp.pad(boxes[:, 1], (0, pad))
    x2 = jnp.pad(boxes[:, 2], (0, pad))
    y2 = jnp.pad(boxes[:, 3], (0, pad))
    ob = jnp.pad(objectness, (0, pad))
    br = jnp.pad(breed_conf, (0, pad))
    out = _run(x1, y1, x2, y2, ob, br)
    return out.reshape(8, OROW)[:5, :MAX_DET].T


# speedup vs baseline: 1.5531x; 1.5531x over previous
import jax
import jax.numpy as jnp
from jax import lax
from jax.experimental import pallas as pl
from jax.experimental.pallas import tpu as pltpu
from jax.experimental.pallas import tpu_sc as plsc

N = 20000
MAX_DET = 300
CONF_THRES = 0.25
IOU_THRES = 0.5

L = 16
NSUB = 16
SLICE = 1264
CH = SLICE // L
NPAD = NSUB * SLICE
KMAX = 4
OROW = 304
OUTW = 8 * OROW
BIG = 3e7
NEG = -1e30


def _sc_nms(x1h, y1h, x2h, y2h, obh, brh, outh,
            sx1, sy1, sx2, sy2, sob, sbr,
            bx1, by1, bx2, by2, bsc, bidx,
            recv, allv, outv, stage):
    c = lax.axis_index("c")
    s = lax.axis_index("s")
    is_writer = jnp.logical_and(c == 0, s == 0)
    base = s * SLICE
    lane = lax.iota(jnp.int32, L)
    lanef = lane.astype(jnp.float32)
    zeros16 = jnp.zeros((L,), jnp.float32)
    neg_inf = jnp.float32(-jnp.inf)

    pltpu.sync_copy(x1h.at[pl.ds(base, SLICE)], sx1)
    pltpu.sync_copy(y1h.at[pl.ds(base, SLICE)], sy1)
    pltpu.sync_copy(x2h.at[pl.ds(base, SLICE)], sx2)
    pltpu.sync_copy(y2h.at[pl.ds(base, SLICE)], sy2)
    pltpu.sync_copy(obh.at[pl.ds(base, SLICE)], sob)
    pltpu.sync_copy(brh.at[pl.ds(base, SLICE)], sbr)

    def _pre(k, carry):
        bsc[pl.ds(k * L, L)] = jnp.full((L,), neg_inf)
        return carry

    lax.fori_loop(0, CH, _pre, 0)

    def _zo(k, carry):
        outv[pl.ds(k * L, L)] = zeros16
        return carry

    lax.fori_loop(0, OUTW // L, _zo, 0)

    basef = base.astype(jnp.float32)

    def _cmp(j, off):
        o = sob[pl.ds(j * L, L)]
        b = sbr[pl.ds(j * L, L)]
        sco = jnp.clip(o, 0.0, 1.0) * jnp.clip(b, 0.0, 1.0)
        alive = sco >= CONF_THRES
        x1v = sx1[pl.ds(j * L, L)]
        y1v = sy1[pl.ds(j * L, L)]
        x2v = sx2[pl.ds(j * L, L)]
        y2v = sy2[pl.ds(j * L, L)]
        idxf = basef + (j * L).astype(jnp.float32) + lanef
        cs = jnp.cumsum(alive.astype(jnp.int32))
        tgt = off + cs - 1
        plsc.store_scatter(bx1, [tgt], x1v, mask=alive)
        plsc.store_scatter(by1, [tgt], y1v, mask=alive)
        plsc.store_scatter(bx2, [tgt], x2v, mask=alive)
        plsc.store_scatter(by2, [tgt], y2v, mask=alive)
        plsc.store_scatter(bsc, [tgt], sco, mask=alive)
        plsc.store_scatter(bidx, [tgt], idxf, mask=alive)
        return off + jnp.max(cs)

    na = lax.fori_loop(0, CH, _cmp, jnp.int32(0))
    nch = (na + (L - 1)) // L

    deg = (jnp.float32(2e9), jnp.float32(2e9),
           jnp.float32(-2e9), jnp.float32(-2e9))

    def _round(carry):
        w, done, rnd, wins = carry

        was = tuple((ww[2] - ww[0]) * (ww[3] - ww[1]) for ww in wins)

        init = (jnp.full((L,), neg_inf), zeros16,
                jnp.full((L,), neg_inf), zeros16)

        def _pass(j, st):
            bs1, bp1, bs2, bp2 = st
            x1v = bx1[pl.ds(j * L, L)]
            y1v = by1[pl.ds(j * L, L)]
            x2v = bx2[pl.ds(j * L, L)]
            y2v = by2[pl.ds(j * L, L)]
            scv = bsc[pl.ds(j * L, L)]
            arv = (x2v - x1v) * (y2v - y1v)
            sup = None
            for t in range(KMAX):
                wx1, wy1, wx2, wy2 = wins[t]
                xx1 = jnp.maximum(wx1, x1v)
                yy1 = jnp.maximum(wy1, y1v)
                xx2 = jnp.minimum(wx2, x2v)
                yy2 = jnp.minimum(wy2, y2v)
                inter = (jnp.maximum(xx2 - xx1, 0.0)
                         * jnp.maximum(yy2 - yy1, 0.0))
                iou = inter / jnp.maximum(was[t] + arv - inter, 1e-9)
                st_ = iou >= IOU_THRES
                sup = st_ if sup is None else jnp.logical_or(sup, st_)
            sc2 = jnp.where(sup, neg_inf, scv)
            bsc[pl.ds(j * L, L)] = sc2
            posv = (j * L).astype(jnp.float32) + lanef
            upd1 = sc2 > bs1
            upd2 = jnp.logical_and(jnp.logical_not(upd1), sc2 > bs2)
            bs2 = jnp.where(upd1, bs1, jnp.where(upd2, sc2, bs2))
            bp2 = jnp.where(upd1, bp1, jnp.where(upd2, posv, bp2))
            bs1 = jnp.where(upd1, sc2, bs1)
            bp1 = jnp.where(upd1, posv, bp1)
            return (bs1, bp1, bs2, bp2)

        bs1, bp1, bs2, bp2 = plsc.parallel_loop(
            0, nch, unroll=4, carry=init)(_pass)

        m1 = jnp.max(bs1)
        p1 = jnp.min(jnp.where(bs1 == m1, bp1, BIG))
        sel1 = jnp.logical_and(bs1 == m1, bp1 == p1)
        c2v = jnp.where(sel1, bs2, bs1)
        c2p = jnp.where(sel1, bp2, bp1)
        m2 = jnp.max(c2v)
        p2 = jnp.min(jnp.where(c2v == m2, c2p, BIG))

        r1 = jnp.full((L,), 0, jnp.int32) + p1.astype(jnp.int32)
        r2 = jnp.full((L,), 0, jnp.int32) + p2.astype(jnp.int32)
        recA = jnp.where(lane == 0, m1,
               jnp.where(lane == 1, plsc.load_gather(bidx, [r1]),
               jnp.where(lane == 2, plsc.load_gather(bx1, [r1]),
               jnp.where(lane == 3, plsc.load_gather(by1, [r1]),
               jnp.where(lane == 4, plsc.load_gather(bx2, [r1]),
               jnp.where(lane == 5, plsc.load_gather(by2, [r1]), 0.0))))))
        recB = jnp.where(lane == 0, m2,
               jnp.where(lane == 1, plsc.load_gather(bidx, [r2]),
               jnp.where(lane == 2, plsc.load_gather(bx1, [r2]),
               jnp.where(lane == 3, plsc.load_gather(by1, [r2]),
               jnp.where(lane == 4, plsc.load_gather(bx2, [r2]),
               jnp.where(lane == 5, plsc.load_gather(by2, [r2]), 0.0))))))
        recv[pl.ds(0, L)] = recA
        recv[pl.ds(L, L)] = recB
        ph = lax.rem(rnd, 2) + 6
        pltpu.sync_copy(recv, stage.at[ph, s])
        plsc.subcore_barrier()
        pltpu.sync_copy(stage.at[ph], allv)

        def g(f):
            return plsc.load_gather(allv, [lane, jnp.full((L,), f, jnp.int32)])

        sA = g(0)
        idA = g(1)
        xA1 = g(2)
        yA1 = g(3)
        xA2 = g(4)
        yA2 = g(5)
        sB = g(16)
        idB = g(17)
        xB1 = g(18)
        yB1 = g(19)
        xB2 = g(20)
        yB2 = g(21)
        arA = (xA2 - xA1) * (yA2 - yA1)
        arB = (xB2 - xB1) * (yB2 - yB1)

        bound = neg_inf
        m0 = jnp.maximum(jnp.max(sA), jnp.max(sB))
        wc = w
        new_wins = []
        for t in range(KMAX):
            pickA = sA >= sB
            sM = jnp.where(pickA, sA, sB)
            idM = jnp.where(pickA, idA, idB)
            xM1 = jnp.where(pickA, xA1, xB1)
            yM1 = jnp.where(pickA, yA1, yB1)
            xM2 = jnp.where(pickA, xA2, xB2)
            yM2 = jnp.where(pickA, yA2, yB2)
            m = jnp.max(sM)
            valid = jnp.logical_and(m > neg_inf, m > bound)
            wi = jnp.min(jnp.where(sM == m, idM, BIG))
            selM = jnp.logical_and(sM == m, idM == wi)
            wx1 = jnp.max(jnp.where(selM, xM1, NEG))
            wy1 = jnp.max(jnp.where(selM, yM1, NEG))
            wx2 = jnp.max(jnp.where(selM, xM2, NEG))
            wy2 = jnp.max(jnp.where(selM, yM2, NEG))
            wa = (wx2 - wx1) * (wy2 - wy1)

            def kio(x1v, y1v, x2v, y2v, arv):
                xx1 = jnp.maximum(wx1, x1v)
                yy1 = jnp.maximum(wy1, y1v)
                xx2 = jnp.minimum(wx2, x2v)
                yy2 = jnp.minimum(wy2, y2v)
                inter = (jnp.maximum(xx2 - xx1, 0.0)
                         * jnp.maximum(yy2 - yy1, 0.0))
                return inter / jnp.maximum(wa + arv - inter, 1e-9)

            supA = jnp.logical_and(kio(xA1, yA1, xA2, yA2, arA) >= IOU_THRES,
                                   valid)
            supB = jnp.logical_and(kio(xB1, yB1, xB2, yB2, arB) >= IOU_THRES,
                                   valid)
            dB = jnp.logical_and(supB, sB > neg_inf)
            bound = jnp.maximum(bound, jnp.max(jnp.where(dB, sB, neg_inf)))
            sA = jnp.where(supA, neg_inf, sA)
            sB = jnp.where(supB, neg_inf, sB)

            wm = jnp.logical_and(
                jnp.logical_and(jnp.logical_and(is_writer, valid), lane == 0),
                wc < MAX_DET)
            mval = jnp.where(valid, m, jnp.float32(0.0))
            ox1 = jnp.where(valid, wx1, jnp.float32(0.0))
            oy1 = jnp.where(valid, wy1, jnp.float32(0.0))
            ox2 = jnp.where(valid, wx2, jnp.float32(0.0))
            oy2 = jnp.where(valid, wy2, jnp.float32(0.0))
            for fld, v in enumerate((ox1, oy1, ox2, oy2, mval)):
                plsc.store_scatter(
                    outv, [jnp.full((L,), fld * OROW, jnp.int32) + wc],
                    v + zeros16, mask=wm)
            wc = wc + valid.astype(jnp.int32)
            new_wins.append((
                jnp.where(valid, wx1, deg[0]),
                jnp.where(valid, wy1, deg[1]),
                jnp.where(valid, wx2, deg[2]),
                jnp.where(valid, wy2, deg[3]),
            ))

        done2 = jnp.logical_not(m0 > neg_inf)
        return (wc, done2, rnd + 1, tuple(new_wins))

    def _cond(carry):
        w, done, rnd, wins = carry
        return jnp.logical_and(w < MAX_DET, jnp.logical_not(done))

    carry0 = (jnp.int32(0), jnp.bool_(False), jnp.int32(0),
              tuple(deg for _ in range(KMAX)))
    lax.while_loop(_cond, _round, carry0)

    @pl.when(is_writer)
    def _flush():
        pltpu.sync_copy(outv, outh)


@jax.jit
def _run(x1, y1, x2, y2, ob, br):
    mesh = plsc.VectorSubcoreMesh(core_axis_name="c", subcore_axis_name="s")
    f = pl.kernel(
        _sc_nms,
        mesh=mesh,
        out_type=jax.ShapeDtypeStruct((OUTW,), jnp.float32),
        compiler_params=pltpu.CompilerParams(needs_layout_passes=False),
        scratch_types=[
            pltpu.VMEM((SLICE,), jnp.float32),
            pltpu.VMEM((SLICE,), jnp.float32),
            pltpu.VMEM((SLICE,), jnp.float32),
            pltpu.VMEM((SLICE,), jnp.float32),
            pltpu.VMEM((SLICE,), jnp.float32),
            pltpu.VMEM((SLICE,), jnp.float32),
            pltpu.VMEM((SLICE,), jnp.float32),
            pltpu.VMEM((SLICE,), jnp.float32),
            pltpu.VMEM((SLICE,), jnp.float32),
            pltpu.VMEM((SLICE,), jnp.float32),
            pltpu.VMEM((SLICE,), jnp.float32),
            pltpu.VMEM((SLICE,), jnp.float32),
            pltpu.VMEM((2 * L,), jnp.float32),
            pltpu.VMEM((NSUB, 2 * L), jnp.float32),
            pltpu.VMEM((OUTW,), jnp.float32),
            pltpu.VMEM_SHARED((8, NSUB, 2 * L), jnp.float32),
        ],
    )
    return f(x1, y1, x2, y2, ob, br)


def kernel(boxes, objectness, breed_conf):
    pad = NPAD - N
    x1 = jnp.pad(boxes[:, 0], (0, pad))
    y1 = jnp.pad(boxes[:, 1], (0, pad))
    x2 = jnp.pad(boxes[:, 2], (0, pad))
    y2 = jnp.pad(boxes[:, 3], (0, pad))
    ob = jnp.pad(objectness, (0, pad))
    br = jnp.pad(breed_conf, (0, pad))
    out = _run(x1, y1, x2, y2, ob, br)
    return out.reshape(8, OROW)[:5, :MAX_DET].T


# speedup vs baseline: 1.6229x; 1.0449x over previous
import jax
import jax.numpy as jnp
from jax import lax
from jax.experimental import pallas as pl
from jax.experimental.pallas import tpu as pltpu
from jax.experimental.pallas import tpu_sc as plsc

N = 20000
MAX_DET = 300
CONF_THRES = 0.25
IOU_THRES = 0.5

L = 16
NSUB = 16
SLICE = 1264
CH = SLICE // L
NPAD = NSUB * SLICE
KMAX = 6
OROW = 304
OUTW = 8 * OROW
BIG = 3e7
NEG = -1e30


def _sc_nms(x1h, y1h, x2h, y2h, obh, brh, outh,
            sx1, sy1, sx2, sy2, sob, sbr,
            bx1, by1, bx2, by2, bsc, bidx,
            recv, allv, outv, stage):
    c = lax.axis_index("c")
    s = lax.axis_index("s")
    is_writer = jnp.logical_and(c == 0, s == 0)
    base = s * SLICE
    lane = lax.iota(jnp.int32, L)
    lanef = lane.astype(jnp.float32)
    zeros16 = jnp.zeros((L,), jnp.float32)
    neg_inf = jnp.float32(-jnp.inf)

    pltpu.sync_copy(x1h.at[pl.ds(base, SLICE)], sx1)
    pltpu.sync_copy(y1h.at[pl.ds(base, SLICE)], sy1)
    pltpu.sync_copy(x2h.at[pl.ds(base, SLICE)], sx2)
    pltpu.sync_copy(y2h.at[pl.ds(base, SLICE)], sy2)
    pltpu.sync_copy(obh.at[pl.ds(base, SLICE)], sob)
    pltpu.sync_copy(brh.at[pl.ds(base, SLICE)], sbr)

    def _pre(k, carry):
        bsc[pl.ds(k * L, L)] = jnp.full((L,), neg_inf)
        return carry

    lax.fori_loop(0, CH, _pre, 0)

    def _zo(k, carry):
        outv[pl.ds(k * L, L)] = zeros16
        return carry

    lax.fori_loop(0, OUTW // L, _zo, 0)

    basef = base.astype(jnp.float32)

    def _cmp(j, off):
        o = sob[pl.ds(j * L, L)]
        b = sbr[pl.ds(j * L, L)]
        sco = jnp.clip(o, 0.0, 1.0) * jnp.clip(b, 0.0, 1.0)
        alive = sco >= CONF_THRES
        x1v = sx1[pl.ds(j * L, L)]
        y1v = sy1[pl.ds(j * L, L)]
        x2v = sx2[pl.ds(j * L, L)]
        y2v = sy2[pl.ds(j * L, L)]
        idxf = basef + (j * L).astype(jnp.float32) + lanef
        cs = jnp.cumsum(alive.astype(jnp.int32))
        tgt = off + cs - 1
        plsc.store_scatter(bx1, [tgt], x1v, mask=alive)
        plsc.store_scatter(by1, [tgt], y1v, mask=alive)
        plsc.store_scatter(bx2, [tgt], x2v, mask=alive)
        plsc.store_scatter(by2, [tgt], y2v, mask=alive)
        plsc.store_scatter(bsc, [tgt], sco, mask=alive)
        plsc.store_scatter(bidx, [tgt], idxf, mask=alive)
        return off + jnp.max(cs)

    na = lax.fori_loop(0, CH, _cmp, jnp.int32(0))
    nch = (na + (L - 1)) // L

    deg = (jnp.float32(2e9), jnp.float32(2e9),
           jnp.float32(-2e9), jnp.float32(-2e9))

    def _round(carry):
        w, done, rnd, wins = carry

        was = tuple((ww[2] - ww[0]) * (ww[3] - ww[1]) for ww in wins)

        init = (jnp.full((L,), neg_inf), zeros16,
                jnp.full((L,), neg_inf), zeros16)

        def _pass(j, st):
            bs1, bp1, bs2, bp2 = st
            x1v = bx1[pl.ds(j * L, L)]
            y1v = by1[pl.ds(j * L, L)]
            x2v = bx2[pl.ds(j * L, L)]
            y2v = by2[pl.ds(j * L, L)]
            scv = bsc[pl.ds(j * L, L)]
            arv = (x2v - x1v) * (y2v - y1v)
            sup = None
            for t in range(KMAX):
                wx1, wy1, wx2, wy2 = wins[t]
                xx1 = jnp.maximum(wx1, x1v)
                yy1 = jnp.maximum(wy1, y1v)
                xx2 = jnp.minimum(wx2, x2v)
                yy2 = jnp.minimum(wy2, y2v)
                inter = (jnp.maximum(xx2 - xx1, 0.0)
                         * jnp.maximum(yy2 - yy1, 0.0))
                iou = inter / jnp.maximum(was[t] + arv - inter, 1e-9)
                st_ = iou >= IOU_THRES
                sup = st_ if sup is None else jnp.logical_or(sup, st_)
            sc2 = jnp.where(sup, neg_inf, scv)
            bsc[pl.ds(j * L, L)] = sc2
            posv = (j * L).astype(jnp.float32) + lanef
            upd1 = sc2 > bs1
            upd2 = jnp.logical_and(jnp.logical_not(upd1), sc2 > bs2)
            bs2 = jnp.where(upd1, bs1, jnp.where(upd2, sc2, bs2))
            bp2 = jnp.where(upd1, bp1, jnp.where(upd2, posv, bp2))
            bs1 = jnp.where(upd1, sc2, bs1)
            bp1 = jnp.where(upd1, posv, bp1)
            return (bs1, bp1, bs2, bp2)

        bs1, bp1, bs2, bp2 = plsc.parallel_loop(
            0, nch, unroll=4, carry=init)(_pass)

        m1 = jnp.max(bs1)
        p1 = jnp.min(jnp.where(bs1 == m1, bp1, BIG))
        sel1 = jnp.logical_and(bs1 == m1, bp1 == p1)
        c2v = jnp.where(sel1, bs2, bs1)
        c2p = jnp.where(sel1, bp2, bp1)
        m2 = jnp.max(c2v)
        p2 = jnp.min(jnp.where(c2v == m2, c2p, BIG))

        r1 = jnp.full((L,), 0, jnp.int32) + p1.astype(jnp.int32)
        r2 = jnp.full((L,), 0, jnp.int32) + p2.astype(jnp.int32)
        recA = jnp.where(lane == 0, m1,
               jnp.where(lane == 1, plsc.load_gather(bidx, [r1]),
               jnp.where(lane == 2, plsc.load_gather(bx1, [r1]),
               jnp.where(lane == 3, plsc.load_gather(by1, [r1]),
               jnp.where(lane == 4, plsc.load_gather(bx2, [r1]),
               jnp.where(lane == 5, plsc.load_gather(by2, [r1]), 0.0))))))
        recB = jnp.where(lane == 0, m2,
               jnp.where(lane == 1, plsc.load_gather(bidx, [r2]),
               jnp.where(lane == 2, plsc.load_gather(bx1, [r2]),
               jnp.where(lane == 3, plsc.load_gather(by1, [r2]),
               jnp.where(lane == 4, plsc.load_gather(bx2, [r2]),
               jnp.where(lane == 5, plsc.load_gather(by2, [r2]), 0.0))))))
        recv[pl.ds(0, L)] = recA
        recv[pl.ds(L, L)] = recB
        ph = lax.rem(rnd, 2) + 6
        pltpu.sync_copy(recv, stage.at[ph, s])
        plsc.subcore_barrier()
        pltpu.sync_copy(stage.at[ph], allv)

        def g(f):
            return plsc.load_gather(allv, [lane, jnp.full((L,), f, jnp.int32)])

        sA = g(0)
        idA = g(1)
        xA1 = g(2)
        yA1 = g(3)
        xA2 = g(4)
        yA2 = g(5)
        sB = g(16)
        idB = g(17)
        xB1 = g(18)
        yB1 = g(19)
        xB2 = g(20)
        yB2 = g(21)
        arA = (xA2 - xA1) * (yA2 - yA1)
        arB = (xB2 - xB1) * (yB2 - yB1)

        bound = neg_inf
        m0 = jnp.maximum(jnp.max(sA), jnp.max(sB))
        wc = w
        new_wins = []
        for t in range(KMAX):
            pickA = sA >= sB
            sM = jnp.where(pickA, sA, sB)
            idM = jnp.where(pickA, idA, idB)
            xM1 = jnp.where(pickA, xA1, xB1)
            yM1 = jnp.where(pickA, yA1, yB1)
            xM2 = jnp.where(pickA, xA2, xB2)
            yM2 = jnp.where(pickA, yA2, yB2)
            m = jnp.max(sM)
            valid = jnp.logical_and(m > neg_inf, m > bound)
            wi = jnp.min(jnp.where(sM == m, idM, BIG))
            selM = jnp.logical_and(sM == m, idM == wi)
            wx1 = jnp.max(jnp.where(selM, xM1, NEG))
            wy1 = jnp.max(jnp.where(selM, yM1, NEG))
            wx2 = jnp.max(jnp.where(selM, xM2, NEG))
            wy2 = jnp.max(jnp.where(selM, yM2, NEG))
            wa = (wx2 - wx1) * (wy2 - wy1)

            def kio(x1v, y1v, x2v, y2v, arv):
                xx1 = jnp.maximum(wx1, x1v)
                yy1 = jnp.maximum(wy1, y1v)
                xx2 = jnp.minimum(wx2, x2v)
                yy2 = jnp.minimum(wy2, y2v)
                inter = (jnp.maximum(xx2 - xx1, 0.0)
                         * jnp.maximum(yy2 - yy1, 0.0))
                return inter / jnp.maximum(wa + arv - inter, 1e-9)

            supA = jnp.logical_and(kio(xA1, yA1, xA2, yA2, arA) >= IOU_THRES,
                                   valid)
            supB = jnp.logical_and(kio(xB1, yB1, xB2, yB2, arB) >= IOU_THRES,
                                   valid)
            dB = jnp.logical_and(supB, sB > neg_inf)
            bound = jnp.maximum(bound, jnp.max(jnp.where(dB, sB, neg_inf)))
            sA = jnp.where(supA, neg_inf, sA)
            sB = jnp.where(supB, neg_inf, sB)

            wm = jnp.logical_and(
                jnp.logical_and(jnp.logical_and(is_writer, valid), lane == 0),
                wc < MAX_DET)
            mval = jnp.where(valid, m, jnp.float32(0.0))
            ox1 = jnp.where(valid, wx1, jnp.float32(0.0))
            oy1 = jnp.where(valid, wy1, jnp.float32(0.0))
            ox2 = jnp.where(valid, wx2, jnp.float32(0.0))
            oy2 = jnp.where(valid, wy2, jnp.float32(0.0))
            for fld, v in enumerate((ox1, oy1, ox2, oy2, mval)):
                plsc.store_scatter(
                    outv, [jnp.full((L,), fld * OROW, jnp.int32) + wc],
                    v + zeros16, mask=wm)
            wc = wc + valid.astype(jnp.int32)
            new_wins.append((
                jnp.where(valid, wx1, deg[0]),
                jnp.where(valid, wy1, deg[1]),
                jnp.where(valid, wx2, deg[2]),
                jnp.where(valid, wy2, deg[3]),
            ))

        done2 = jnp.logical_not(m0 > neg_inf)
        return (wc, done2, rnd + 1, tuple(new_wins))

    def _cond(carry):
        w, done, rnd, wins = carry
        return jnp.logical_and(w < MAX_DET, jnp.logical_not(done))

    carry0 = (jnp.int32(0), jnp.bool_(False), jnp.int32(0),
              tuple(deg for _ in range(KMAX)))
    lax.while_loop(_cond, _round, carry0)

    @pl.when(is_writer)
    def _flush():
        pltpu.sync_copy(outv, outh)


@jax.jit
def _run(x1, y1, x2, y2, ob, br):
    mesh = plsc.VectorSubcoreMesh(core_axis_name="c", subcore_axis_name="s")
    f = pl.kernel(
        _sc_nms,
        mesh=mesh,
        out_type=jax.ShapeDtypeStruct((OUTW,), jnp.float32),
        compiler_params=pltpu.CompilerParams(needs_layout_passes=False),
        scratch_types=[
            pltpu.VMEM((SLICE,), jnp.float32),
            pltpu.VMEM((SLICE,), jnp.float32),
            pltpu.VMEM((SLICE,), jnp.float32),
            pltpu.VMEM((SLICE,), jnp.float32),
            pltpu.VMEM((SLICE,), jnp.float32),
            pltpu.VMEM((SLICE,), jnp.float32),
            pltpu.VMEM((SLICE,), jnp.float32),
            pltpu.VMEM((SLICE,), jnp.float32),
            pltpu.VMEM((SLICE,), jnp.float32),
            pltpu.VMEM((SLICE,), jnp.float32),
            pltpu.VMEM((SLICE,), jnp.float32),
            pltpu.VMEM((SLICE,), jnp.float32),
            pltpu.VMEM((2 * L,), jnp.float32),
            pltpu.VMEM((NSUB, 2 * L), jnp.float32),
            pltpu.VMEM((OUTW,), jnp.float32),
            pltpu.VMEM_SHARED((8, NSUB, 2 * L), jnp.float32),
        ],
    )
    return f(x1, y1, x2, y2, ob, br)


def kernel(boxes, objectness, breed_conf):
    pad = NPAD - N
    x1 = jnp.pad(boxes[:, 0], (0, pad))
    y1 = jnp.pad(boxes[:, 1], (0, pad))
    x2 = jnp.pad(boxes[:, 2], (0, pad))
    y2 = jnp.pad(boxes[:, 3], (0, pad))
    ob = jnp.pad(objectness, (0, pad))
    br = jnp.pad(breed_conf, (0, pad))
    out = _run(x1, y1, x2, y2, ob, br)
    return out.reshape(8, OROW)[:5, :MAX_DET].T
